# baseline (speedup 1.0000x reference)
import functools

import jax
import jax.numpy as jnp
from jax import lax
from jax.experimental import pallas as pl
from jax.experimental.pallas import tpu as pltpu
from jax.experimental.pallas import tpu_sc as plsc

_N = 1024
_THR = 6.0
_NC = 2
_NS = 16
_L = 16
_CHUNKS = _N // _L


def _min_argmin_axis0(P):
    P3 = P.reshape(_N // 8, 8, _N)

    av = P3[0]
    ai = jnp.zeros((8, _N), jnp.int32)
    for i in range(1, _N // 8):
        sl = P3[i]
        m = sl < av
        av = jnp.minimum(av, sl)
        ai = jnp.where(m, i, ai)

    n8 = ai * 8 + lax.broadcasted_iota(jnp.int32, (8, _N), 0)
    v = jnp.min(av, axis=0)
    idx = jnp.min(jnp.where(av == v[None, :], n8, jnp.int32(_N)), axis=0)
    return v, idx


def _tc_minargmin_kernel(a_ref, b_ref, mins_ref, idxs_ref):
    A = a_ref[0]
    B = b_ref[0]
    P = lax.dot_general(A, B, (((1,), (1,)), ((), ())),
                        preferred_element_type=jnp.float32)
    PT = lax.dot_general(B, A, (((1,), (1,)), ((), ())),
                         preferred_element_type=jnp.float32)

    min_c, idx_c = _min_argmin_axis0(P)
    min_r, idx_r = _min_argmin_axis0(PT)

    mins_ref[0, 0, :] = min_r
    mins_ref[0, 1, :] = min_c
    idxs_ref[0, 0, :] = idx_r
    idxs_ref[0, 1, :] = idx_c


def _sqrt16(x):
    i = lax.bitcast_convert_type(x, jnp.int32)
    y = lax.bitcast_convert_type(
        jnp.int32(0x1FBD1DF5) + (i >> 1), jnp.float32)
    for _ in range(4):
        y = 0.5 * (y + x / y)
    return y


def _sc_one_direction(mind_v, idx_v, d, dstx, dsty, srcx, srcy,
                      count_v, dist_v):
    ones = jnp.ones((_L,), jnp.float32)
    zeros = jnp.zeros((_L,), jnp.float32)

    def zero_body(j, carry):
        count_v[pl.ds(j * _L, _L)] = zeros
        return carry

    lax.fori_loop(0, _CHUNKS, zero_body, 0, unroll=8)

    def hist_body(j, carry):
        sl = pl.ds(j * _L, _L)
        vi = idx_v[d, sl]
        pfd = mind_v[d, sl] <= _THR
        plsc.addupdate_scatter(count_v, [vi], ones, mask=pfd)
        sx = plsc.load_gather(srcx, [vi])
        sy = plsc.load_gather(srcy, [vi])
        ddx = dstx[sl] - sx
        ddy = dsty[sl] - sy
        dist = _sqrt16(ddx * ddx + ddy * ddy)
        dist_v[sl] = jnp.where(pfd, dist, 0.0)
        return carry

    lax.fori_loop(0, _CHUNKS, hist_body, 0, unroll=4)

    def sum_body(j, carry):
        s_acc, g_acc, m_acc = carry
        sl = pl.ds(j * _L, _L)
        vi = idx_v[d, sl]
        pfd = mind_v[d, sl] <= _THR
        wv = plsc.load_gather(count_v, [vi])
        s_acc = s_acc + jnp.where(pfd, dist_v[sl] / (wv + 1e-6), 0.0)
        cnt = count_v[sl]
        g_acc = g_acc + jnp.where(cnt > 0.5, 1.0, 0.0)
        m_acc = m_acc + cnt
        return s_acc, g_acc, m_acc

    s_acc, g_acc, m_acc = lax.fori_loop(
        0, _CHUNKS, sum_body, (zeros, zeros, zeros), unroll=4)

    s_v = jnp.full((_L,), jnp.sum(s_acc, axis=0), jnp.float32)
    g_v = jnp.full((_L,), jnp.sum(g_acc, axis=0), jnp.float32)
    m_v = jnp.full((_L,), jnp.sum(m_acc, axis=0), jnp.float32)
    unm = jnp.where(m_v < _N - 0.5, 1.0, 0.0)
    n_groups = jnp.maximum(g_v + unm, 1.0)
    return -(s_v + unm) / n_groups


def _sc_reward_body(mins_hbm, idxs_hbm, sxy_hbm, gxy_hbm, out_hbm,
                    mind_v, idx_v, sx_v, sy_v, gx_v, gy_v,
                    count_v, dist_v, row_v):
    wid = lax.axis_index("s") * _NC + lax.axis_index("c")
    acc = jnp.zeros((_L,), jnp.float32)
    for v in range(2):
        bv = wid * 2 + v
        pltpu.sync_copy(mins_hbm.at[bv], mind_v)
        pltpu.sync_copy(idxs_hbm.at[bv], idx_v)
        pltpu.sync_copy(sxy_hbm.at[bv, 0], sx_v)
        pltpu.sync_copy(sxy_hbm.at[bv, 1], sy_v)
        pltpu.sync_copy(gxy_hbm.at[bv, 0], gx_v)
        pltpu.sync_copy(gxy_hbm.at[bv, 1], gy_v)
        for d in range(2):
            if d == 0:
                args = (sx_v, sy_v, gx_v, gy_v)
            else:
                args = (gx_v, gy_v, sx_v, sy_v)
            acc = acc + _sc_one_direction(mind_v, idx_v, d, *args,
                                          count_v, dist_v)
    row_v[...] = acc * 0.25
    pltpu.sync_copy(row_v, out_hbm.at[wid])


@jax.jit
def kernel(achieved_goal, desired_goal, norm_mean, norm_std):
    state = achieved_goal * norm_std + norm_mean
    goal = desired_goal * norm_std + norm_mean
    bs, n_views, n_particles, nfeat = state.shape
    bv = bs * n_views

    sv = state[..., 5:9].reshape(bv, n_particles, 4)
    gv = goal[..., 5:9].reshape(bv, n_particles, 4)
    sxy_t = state[..., :2].reshape(bv, n_particles, 2).transpose(0, 2, 1)
    gxy_t = goal[..., :2].reshape(bv, n_particles, 2).transpose(0, 2, 1)

    xx = jnp.sum(sv * sv, axis=-1, keepdims=True)
    yy = jnp.sum(gv * gv, axis=-1, keepdims=True)
    ones = jnp.ones_like(xx)
    zeros2 = jnp.zeros(xx.shape[:-1] + (2,), jnp.float32)
    A = jnp.concatenate([-2.0 * sv, xx, ones, zeros2], axis=-1)
    B = jnp.concatenate([gv, ones, yy, zeros2], axis=-1)

    mins, idxs = pl.pallas_call(
        _tc_minargmin_kernel,
        grid=(bv,),
        compiler_params=pltpu.CompilerParams(
            dimension_semantics=("parallel",)),
        in_specs=[
            pl.BlockSpec((1, n_particles, 8), lambda i: (i, 0, 0)),
            pl.BlockSpec((1, n_particles, 8), lambda i: (i, 0, 0)),
        ],
        out_specs=[
            pl.BlockSpec((1, 2, n_particles), lambda i: (i, 0, 0)),
            pl.BlockSpec((1, 2, n_particles), lambda i: (i, 0, 0)),
        ],
        out_shape=[
            jax.ShapeDtypeStruct((bv, 2, n_particles), jnp.float32),
            jax.ShapeDtypeStruct((bv, 2, n_particles), jnp.int32),
        ],
    )(A, B)

    sc_fn = pl.kernel(
        _sc_reward_body,
        out_type=jax.ShapeDtypeStruct((bs, _L), jnp.float32),
        mesh=plsc.VectorSubcoreMesh(core_axis_name="c", subcore_axis_name="s",
                                    num_cores=_NC, num_subcores=_NS),
        compiler_params=pltpu.CompilerParams(needs_layout_passes=False),
        scratch_types=[
            pltpu.VMEM((2, n_particles), jnp.float32),
            pltpu.VMEM((2, n_particles), jnp.int32),
            pltpu.VMEM((n_particles,), jnp.float32),
            pltpu.VMEM((n_particles,), jnp.float32),
            pltpu.VMEM((n_particles,), jnp.float32),
            pltpu.VMEM((n_particles,), jnp.float32),
            pltpu.VMEM((n_particles,), jnp.float32),
            pltpu.VMEM((n_particles,), jnp.float32),
            pltpu.VMEM((_L,), jnp.float32),
        ],
    )
    sample_rewards = sc_fn(mins, idxs, sxy_t, gxy_t)

    return sample_rewards[:, 0][:, None]


# speedup vs baseline: 1.9300x; 1.9300x over previous
import functools

import jax
import jax.numpy as jnp
from jax import lax
from jax.experimental import pallas as pl
from jax.experimental.pallas import tpu as pltpu
from jax.experimental.pallas import tpu_sc as plsc

_N = 1024
_THR = 6.0
_NC = 2
_NS = 16
_L = 16
_CHUNKS = _N // _L


def _min_argmin_axis0(P):
    P3 = P.reshape(_N // 8, 8, _N)

    av = P3[0]
    ai = jnp.zeros((8, _N), jnp.int32)
    for i in range(1, _N // 8):
        sl = P3[i]
        m = sl < av
        av = jnp.minimum(av, sl)
        ai = jnp.where(m, i, ai)

    n8 = ai * 8 + lax.broadcasted_iota(jnp.int32, (8, _N), 0)
    v = jnp.min(av, axis=0)
    idx = jnp.min(jnp.where(av == v[None, :], n8, jnp.int32(_N)), axis=0)
    return v, idx


def _tc_minargmin_kernel(s_ref, g_ref, std_ref, mean_ref,
                         mins_ref, idxs_ref, sxyt_ref, gxyt_ref):
    s = s_ref[0] * std_ref[...] + mean_ref[...]
    g = g_ref[0] * std_ref[...] + mean_ref[...]

    lane = lax.broadcasted_iota(jnp.int32, (1, s.shape[1]), 1)
    vis = jnp.where((lane >= 5) & (lane < 9), 1.0, 0.0)
    sv = s * vis
    gv = g * vis

    xx = jnp.sum(sv * sv, axis=-1)[:, None]
    yy = jnp.sum(gv * gv, axis=-1)[:, None]
    ones = jnp.ones((_N, 1), jnp.float32)
    A = jnp.concatenate([-2.0 * sv, xx, ones], axis=1)
    B = jnp.concatenate([gv, ones, yy], axis=1)
    P = lax.dot_general(A, B, (((1,), (1,)), ((), ())),
                        preferred_element_type=jnp.float32)
    PT = lax.dot_general(B, A, (((1,), (1,)), ((), ())),
                         preferred_element_type=jnp.float32)

    min_c, idx_c = _min_argmin_axis0(P)
    min_r, idx_r = _min_argmin_axis0(PT)

    mins_ref[0, 0, :] = min_r
    mins_ref[0, 1, :] = min_c
    idxs_ref[0, 0, :] = idx_r
    idxs_ref[0, 1, :] = idx_c
    sxyt_ref[0] = s[:, 0:2].T
    gxyt_ref[0] = g[:, 0:2].T


def _sqrt16(x):
    i = lax.bitcast_convert_type(x, jnp.int32)
    y = lax.bitcast_convert_type(
        jnp.int32(0x1FBD1DF5) + (i >> 1), jnp.float32)
    for _ in range(4):
        y = 0.5 * (y + x / y)
    return y


def _sc_one_direction(mind_v, idx_v, d, dstx, dsty, srcx, srcy,
                      count_v, dist_v):
    ones = jnp.ones((_L,), jnp.float32)
    zeros = jnp.zeros((_L,), jnp.float32)

    def zero_body(j, carry):
        count_v[pl.ds(j * _L, _L)] = zeros
        return carry

    lax.fori_loop(0, _CHUNKS, zero_body, 0, unroll=8)

    def hist_body(j, carry):
        sl = pl.ds(j * _L, _L)
        vi = idx_v[d, sl]
        pfd = mind_v[d, sl] <= _THR
        plsc.addupdate_scatter(count_v, [vi], ones, mask=pfd)
        sx = plsc.load_gather(srcx, [vi])
        sy = plsc.load_gather(srcy, [vi])
        ddx = dstx[sl] - sx
        ddy = dsty[sl] - sy
        dist = _sqrt16(ddx * ddx + ddy * ddy)
        dist_v[sl] = jnp.where(pfd, dist, 0.0)
        return carry

    lax.fori_loop(0, _CHUNKS, hist_body, 0, unroll=4)

    def sum_body(j, carry):
        s_acc, g_acc, m_acc = carry
        sl = pl.ds(j * _L, _L)
        vi = idx_v[d, sl]
        pfd = mind_v[d, sl] <= _THR
        wv = plsc.load_gather(count_v, [vi])
        s_acc = s_acc + jnp.where(pfd, dist_v[sl] / (wv + 1e-6), 0.0)
        cnt = count_v[sl]
        g_acc = g_acc + jnp.where(cnt > 0.5, 1.0, 0.0)
        m_acc = m_acc + cnt
        return s_acc, g_acc, m_acc

    s_acc, g_acc, m_acc = lax.fori_loop(
        0, _CHUNKS, sum_body, (zeros, zeros, zeros), unroll=4)

    s_v = jnp.full((_L,), jnp.sum(s_acc, axis=0), jnp.float32)
    g_v = jnp.full((_L,), jnp.sum(g_acc, axis=0), jnp.float32)
    m_v = jnp.full((_L,), jnp.sum(m_acc, axis=0), jnp.float32)
    unm = jnp.where(m_v < _N - 0.5, 1.0, 0.0)
    n_groups = jnp.maximum(g_v + unm, 1.0)
    return -(s_v + unm) / n_groups


def _sc_reward_body(mins_hbm, idxs_hbm, sxy_hbm, gxy_hbm, out_hbm,
                    mind_v, idx_v, sx_v, sy_v, gx_v, gy_v,
                    count_v, dist_v, row_v):
    wid = lax.axis_index("s") * _NC + lax.axis_index("c")
    acc = jnp.zeros((_L,), jnp.float32)
    for v in range(2):
        bv = wid * 2 + v
        pltpu.sync_copy(mins_hbm.at[bv], mind_v)
        pltpu.sync_copy(idxs_hbm.at[bv], idx_v)
        pltpu.sync_copy(sxy_hbm.at[bv, 0], sx_v)
        pltpu.sync_copy(sxy_hbm.at[bv, 1], sy_v)
        pltpu.sync_copy(gxy_hbm.at[bv, 0], gx_v)
        pltpu.sync_copy(gxy_hbm.at[bv, 1], gy_v)
        for d in range(2):
            if d == 0:
                args = (sx_v, sy_v, gx_v, gy_v)
            else:
                args = (gx_v, gy_v, sx_v, sy_v)
            acc = acc + _sc_one_direction(mind_v, idx_v, d, *args,
                                          count_v, dist_v)
    row_v[...] = acc * 0.25
    pltpu.sync_copy(row_v, out_hbm.at[wid])


@jax.jit
def kernel(achieved_goal, desired_goal, norm_mean, norm_std):
    bs, n_views, n_particles, nfeat = achieved_goal.shape
    bv = bs * n_views

    s_raw = achieved_goal.reshape(bv, n_particles, nfeat)
    g_raw = desired_goal.reshape(bv, n_particles, nfeat)
    std2 = norm_std.reshape(1, nfeat)
    mean2 = norm_mean.reshape(1, nfeat)

    mins, idxs, sxy_t, gxy_t = pl.pallas_call(
        _tc_minargmin_kernel,
        grid=(bv,),
        compiler_params=pltpu.CompilerParams(
            dimension_semantics=("parallel",)),
        in_specs=[
            pl.BlockSpec((1, n_particles, nfeat), lambda i: (i, 0, 0)),
            pl.BlockSpec((1, n_particles, nfeat), lambda i: (i, 0, 0)),
            pl.BlockSpec((1, nfeat), lambda i: (0, 0)),
            pl.BlockSpec((1, nfeat), lambda i: (0, 0)),
        ],
        out_specs=[
            pl.BlockSpec((1, 2, n_particles), lambda i: (i, 0, 0)),
            pl.BlockSpec((1, 2, n_particles), lambda i: (i, 0, 0)),
            pl.BlockSpec((1, 2, n_particles), lambda i: (i, 0, 0)),
            pl.BlockSpec((1, 2, n_particles), lambda i: (i, 0, 0)),
        ],
        out_shape=[
            jax.ShapeDtypeStruct((bv, 2, n_particles), jnp.float32),
            jax.ShapeDtypeStruct((bv, 2, n_particles), jnp.int32),
            jax.ShapeDtypeStruct((bv, 2, n_particles), jnp.float32),
            jax.ShapeDtypeStruct((bv, 2, n_particles), jnp.float32),
        ],
    )(s_raw, g_raw, std2, mean2)

    sc_fn = pl.kernel(
        _sc_reward_body,
        out_type=jax.ShapeDtypeStruct((bs, _L), jnp.float32),
        mesh=plsc.VectorSubcoreMesh(core_axis_name="c", subcore_axis_name="s",
                                    num_cores=_NC, num_subcores=_NS),
        compiler_params=pltpu.CompilerParams(needs_layout_passes=False),
        scratch_types=[
            pltpu.VMEM((2, n_particles), jnp.float32),
            pltpu.VMEM((2, n_particles), jnp.int32),
            pltpu.VMEM((n_particles,), jnp.float32),
            pltpu.VMEM((n_particles,), jnp.float32),
            pltpu.VMEM((n_particles,), jnp.float32),
            pltpu.VMEM((n_particles,), jnp.float32),
            pltpu.VMEM((n_particles,), jnp.float32),
            pltpu.VMEM((n_particles,), jnp.float32),
            pltpu.VMEM((_L,), jnp.float32),
        ],
    )
    sample_rewards = sc_fn(mins, idxs, sxy_t, gxy_t)

    return sample_rewards[:, 0][:, None]


# speedup vs baseline: 2.0071x; 1.0399x over previous
import functools

import jax
import jax.numpy as jnp
from jax import lax
from jax.experimental import pallas as pl
from jax.experimental.pallas import tpu as pltpu
from jax.experimental.pallas import tpu_sc as plsc

_N = 1024
_THR = 6.0
_NC = 2
_NS = 16
_L = 16
_CHUNKS = _N // _L


def _min_argmin_axis0(P):
    P3 = P.reshape(_N // 8, 8, _N)

    av = P3[0]
    ai = jnp.zeros((8, _N), jnp.int32)
    for i in range(1, _N // 8):
        sl = P3[i]
        m = sl < av
        av = jnp.minimum(av, sl)
        ai = jnp.where(m, i, ai)

    n8 = ai * 8 + lax.broadcasted_iota(jnp.int32, (8, _N), 0)
    v = jnp.min(av, axis=0)
    idx = jnp.min(jnp.where(av == v[None, :], n8, jnp.int32(_N)), axis=0)
    return v, idx


def _tc_minargmin_kernel(s_ref, g_ref, std_ref, mean_ref,
                         mins_ref, idxs_ref, sxyt_ref, gxyt_ref):
    for b in range(s_ref.shape[0]):
        _tc_one_pair(s_ref, g_ref, std_ref, mean_ref,
                     mins_ref, idxs_ref, sxyt_ref, gxyt_ref, b)


def _tc_one_pair(s_ref, g_ref, std_ref, mean_ref,
                 mins_ref, idxs_ref, sxyt_ref, gxyt_ref, b):
    s = s_ref[b] * std_ref[...] + mean_ref[...]
    g = g_ref[b] * std_ref[...] + mean_ref[...]

    lane = lax.broadcasted_iota(jnp.int32, (1, s.shape[1]), 1)
    vis = jnp.where((lane >= 5) & (lane < 9), 1.0, 0.0)
    sv = s * vis
    gv = g * vis

    xx = jnp.sum(sv * sv, axis=-1)[:, None]
    yy = jnp.sum(gv * gv, axis=-1)[:, None]
    ones = jnp.ones((_N, 1), jnp.float32)
    A = jnp.concatenate([-2.0 * sv, xx, ones], axis=1)
    B = jnp.concatenate([gv, ones, yy], axis=1)
    P = lax.dot_general(A, B, (((1,), (1,)), ((), ())),
                        preferred_element_type=jnp.float32)
    PT = lax.dot_general(B, A, (((1,), (1,)), ((), ())),
                         preferred_element_type=jnp.float32)

    min_c, idx_c = _min_argmin_axis0(P)
    min_r, idx_r = _min_argmin_axis0(PT)

    mins_ref[b, 0, :] = min_r
    mins_ref[b, 1, :] = min_c
    idxs_ref[b, 0, :] = idx_r
    idxs_ref[b, 1, :] = idx_c
    sxyt_ref[b] = s[:, 0:2].T
    gxyt_ref[b] = g[:, 0:2].T


def _sqrt16(x):
    i = lax.bitcast_convert_type(x, jnp.int32)
    y = lax.bitcast_convert_type(
        jnp.int32(0x1FBD1DF5) + (i >> 1), jnp.float32)
    for _ in range(4):
        y = 0.5 * (y + x / y)
    return y


def _sc_one_direction(mind_v, idx_v, d, dstx, dsty, srcx, srcy,
                      count_v, dist_v):
    ones = jnp.ones((_L,), jnp.float32)
    zeros = jnp.zeros((_L,), jnp.float32)

    def zero_body(j, carry):
        count_v[pl.ds(j * _L, _L)] = zeros
        return carry

    lax.fori_loop(0, _CHUNKS, zero_body, 0, unroll=8)

    def hist_body(j, carry):
        sl = pl.ds(j * _L, _L)
        vi = idx_v[d, sl]
        pfd = mind_v[d, sl] <= _THR
        plsc.addupdate_scatter(count_v, [vi], ones, mask=pfd)
        sx = plsc.load_gather(srcx, [vi])
        sy = plsc.load_gather(srcy, [vi])
        ddx = dstx[sl] - sx
        ddy = dsty[sl] - sy
        dist = _sqrt16(ddx * ddx + ddy * ddy)
        dist_v[sl] = jnp.where(pfd, dist, 0.0)
        return carry

    lax.fori_loop(0, _CHUNKS, hist_body, 0, unroll=4)

    def sum_body(j, carry):
        s_acc, g_acc, m_acc = carry
        sl = pl.ds(j * _L, _L)
        vi = idx_v[d, sl]
        pfd = mind_v[d, sl] <= _THR
        wv = plsc.load_gather(count_v, [vi])
        s_acc = s_acc + jnp.where(pfd, dist_v[sl] / (wv + 1e-6), 0.0)
        cnt = count_v[sl]
        g_acc = g_acc + jnp.where(cnt > 0.5, 1.0, 0.0)
        m_acc = m_acc + cnt
        return s_acc, g_acc, m_acc

    s_acc, g_acc, m_acc = lax.fori_loop(
        0, _CHUNKS, sum_body, (zeros, zeros, zeros), unroll=4)

    s_v = jnp.full((_L,), jnp.sum(s_acc, axis=0), jnp.float32)
    g_v = jnp.full((_L,), jnp.sum(g_acc, axis=0), jnp.float32)
    m_v = jnp.full((_L,), jnp.sum(m_acc, axis=0), jnp.float32)
    unm = jnp.where(m_v < _N - 0.5, 1.0, 0.0)
    n_groups = jnp.maximum(g_v + unm, 1.0)
    return -(s_v + unm) / n_groups


def _sc_reward_body(mins_hbm, idxs_hbm, sxy_hbm, gxy_hbm, out_hbm,
                    mind_v, idx_v, sx_v, sy_v, gx_v, gy_v,
                    count_v, dist_v, row_v):
    wid = lax.axis_index("s") * _NC + lax.axis_index("c")
    acc = jnp.zeros((_L,), jnp.float32)
    for v in range(2):
        bv = wid * 2 + v
        pltpu.sync_copy(mins_hbm.at[bv], mind_v)
        pltpu.sync_copy(idxs_hbm.at[bv], idx_v)
        pltpu.sync_copy(sxy_hbm.at[bv, 0], sx_v)
        pltpu.sync_copy(sxy_hbm.at[bv, 1], sy_v)
        pltpu.sync_copy(gxy_hbm.at[bv, 0], gx_v)
        pltpu.sync_copy(gxy_hbm.at[bv, 1], gy_v)
        for d in range(2):
            if d == 0:
                args = (sx_v, sy_v, gx_v, gy_v)
            else:
                args = (gx_v, gy_v, sx_v, sy_v)
            acc = acc + _sc_one_direction(mind_v, idx_v, d, *args,
                                          count_v, dist_v)
    row_v[...] = acc * 0.25
    pltpu.sync_copy(row_v, out_hbm.at[wid])


@jax.jit
def kernel(achieved_goal, desired_goal, norm_mean, norm_std):
    bs, n_views, n_particles, nfeat = achieved_goal.shape
    bv = bs * n_views

    s_raw = achieved_goal.reshape(bv, n_particles, nfeat)
    g_raw = desired_goal.reshape(bv, n_particles, nfeat)
    std2 = norm_std.reshape(1, nfeat)
    mean2 = norm_mean.reshape(1, nfeat)

    mins, idxs, sxy_t, gxy_t = pl.pallas_call(
        _tc_minargmin_kernel,
        grid=(bv // 4,),
        compiler_params=pltpu.CompilerParams(
            dimension_semantics=("parallel",)),
        in_specs=[
            pl.BlockSpec((4, n_particles, nfeat), lambda i: (i, 0, 0)),
            pl.BlockSpec((4, n_particles, nfeat), lambda i: (i, 0, 0)),
            pl.BlockSpec((1, nfeat), lambda i: (0, 0)),
            pl.BlockSpec((1, nfeat), lambda i: (0, 0)),
        ],
        out_specs=[
            pl.BlockSpec((4, 2, n_particles), lambda i: (i, 0, 0)),
            pl.BlockSpec((4, 2, n_particles), lambda i: (i, 0, 0)),
            pl.BlockSpec((4, 2, n_particles), lambda i: (i, 0, 0)),
            pl.BlockSpec((4, 2, n_particles), lambda i: (i, 0, 0)),
        ],
        out_shape=[
            jax.ShapeDtypeStruct((bv, 2, n_particles), jnp.float32),
            jax.ShapeDtypeStruct((bv, 2, n_particles), jnp.int32),
            jax.ShapeDtypeStruct((bv, 2, n_particles), jnp.float32),
            jax.ShapeDtypeStruct((bv, 2, n_particles), jnp.float32),
        ],
    )(s_raw, g_raw, std2, mean2)

    sc_fn = pl.kernel(
        _sc_reward_body,
        out_type=jax.ShapeDtypeStruct((bs, _L), jnp.float32),
        mesh=plsc.VectorSubcoreMesh(core_axis_name="c", subcore_axis_name="s",
                                    num_cores=_NC, num_subcores=_NS),
        compiler_params=pltpu.CompilerParams(needs_layout_passes=False),
        scratch_types=[
            pltpu.VMEM((2, n_particles), jnp.float32),
            pltpu.VMEM((2, n_particles), jnp.int32),
            pltpu.VMEM((n_particles,), jnp.float32),
            pltpu.VMEM((n_particles,), jnp.float32),
            pltpu.VMEM((n_particles,), jnp.float32),
            pltpu.VMEM((n_particles,), jnp.float32),
            pltpu.VMEM((n_particles,), jnp.float32),
            pltpu.VMEM((n_particles,), jnp.float32),
            pltpu.VMEM((_L,), jnp.float32),
        ],
    )
    sample_rewards = sc_fn(mins, idxs, sxy_t, gxy_t)

    return sample_rewards[:, 0][:, None]
